# baseline (speedup 1.0000x reference)
import functools

import jax
import jax.numpy as jnp
from jax import lax
from jax.experimental import pallas as pl
from jax.experimental.pallas import tpu as pltpu
from jax.experimental.pallas import tpu_sc as plsc

VOCAB = 100000
EMBED_DIM = 300
D_PAD = 304
BATCH = 4096
SEQ_LEN = 200

_B = BATCH * SEQ_LEN
_NW = 32
_B_PER_W = _B // _NW
_CHUNK = 64
_N_CHUNKS = _B_PER_W // _CHUNK
_CHUNK_WORDS = _CHUNK * EMBED_DIM
_LANES = 16


def _make_sc_gather():
    mesh = plsc.VectorSubcoreMesh(core_axis_name="c", subcore_axis_name="s")

    @functools.partial(
        pl.kernel,
        mesh=mesh,
        out_type=jax.ShapeDtypeStruct((_B * EMBED_DIM,), jnp.float32),
        compiler_params=pltpu.CompilerParams(use_tc_tiling_on_sc=False),
        scratch_types=[
            pltpu.VMEM((_N_CHUNKS, _CHUNK), jnp.int32),
            pltpu.VMEM((_CHUNK, D_PAD), jnp.float32),
            pltpu.VMEM((_CHUNK, D_PAD), jnp.float32),
            pltpu.VMEM((_CHUNK_WORDS,), jnp.float32),
            pltpu.VMEM((_CHUNK_WORDS,), jnp.float32),
            pltpu.SemaphoreType.DMA,
            pltpu.SemaphoreType.DMA,
            pltpu.SemaphoreType.DMA,
            pltpu.SemaphoreType.DMA,
        ],
    )
    def k(idx_hbm, table_hbm, out_hbm, idx_v, buf0, buf1, cb0, cb1,
          sem0, sem1, osem0, osem1):
        wid = lax.axis_index("s") * 2 + lax.axis_index("c")
        obase = wid * _B_PER_W * EMBED_DIM

        pltpu.sync_copy(idx_hbm.at[wid], idx_v)

        bufs = (buf0, buf1)
        cbs = (cb0, cb1)
        sems = (sem0, sem1)
        osems = (osem0, osem1)

        pltpu.async_copy(table_hbm.at[idx_v.at[0]], buf0, sem0)

        def compact(src, dst):
            def row(r, carry):
                o = r * EMBED_DIM
                for kk in range(EMBED_DIM // _LANES):
                    v = src[r, pl.ds(kk * _LANES, _LANES)]
                    dst[pl.ds(o + kk * _LANES, _LANES)] = v
                v = src[r, pl.ds(EMBED_DIM - _LANES, _LANES)]
                dst[pl.ds(o + EMBED_DIM - _LANES, _LANES)] = v
                return carry

            lax.fori_loop(0, _CHUNK, row, 0)

        def outer(i, carry):
            for p in range(2):
                j = i * 2 + p
                cur, cur_sem = bufs[p], sems[p]
                cb, osem = cbs[p], osems[p]
                nxt, nxt_sem = bufs[1 - p], sems[1 - p]

                @pl.when(j + 1 < _N_CHUNKS)
                def _():
                    pltpu.async_copy(table_hbm.at[idx_v.at[j + 1]], nxt, nxt_sem)

                pltpu.make_async_copy(table_hbm.at[idx_v.at[j]], cur, cur_sem).wait()

                @pl.when(j >= 2)
                def _():
                    pltpu.make_async_copy(
                        cb, out_hbm.at[pl.ds(0, _CHUNK_WORDS)], osem
                    ).wait()

                compact(cur, cb)
                pltpu.async_copy(
                    cb,
                    out_hbm.at[pl.ds(obase + j * _CHUNK_WORDS, _CHUNK_WORDS)],
                    osem,
                )
            return carry

        lax.fori_loop(0, _N_CHUNKS // 2, outer, 0)

        pltpu.make_async_copy(cb0, out_hbm.at[pl.ds(0, _CHUNK_WORDS)], osem0).wait()
        pltpu.make_async_copy(cb1, out_hbm.at[pl.ds(0, _CHUNK_WORDS)], osem1).wait()

    return k


_sc_gather = _make_sc_gather()


def kernel(indices, table):
    idx = indices.reshape(_NW, _N_CHUNKS, _CHUNK).astype(jnp.int32)
    table_pad = jnp.pad(table, ((0, 0), (0, D_PAD - EMBED_DIM)))
    out = _sc_gather(idx, table_pad)
    return out.reshape(BATCH, SEQ_LEN, EMBED_DIM)


# speedup vs baseline: 1.0011x; 1.0011x over previous
import functools

import jax
import jax.numpy as jnp
from jax import lax
from jax.experimental import pallas as pl
from jax.experimental.pallas import tpu as pltpu
from jax.experimental.pallas import tpu_sc as plsc

VOCAB = 100000
EMBED_DIM = 300
D_PAD = 304
BATCH = 4096
SEQ_LEN = 200

_B = BATCH * SEQ_LEN
_NW = 32
_B_PER_W = _B // _NW
_CHUNK = 64
_N_CHUNKS = _B_PER_W // _CHUNK
_CHUNK_WORDS = _CHUNK * EMBED_DIM
_LANES = 16


def _make_sc_gather():
    mesh = plsc.VectorSubcoreMesh(core_axis_name="c", subcore_axis_name="s")

    @functools.partial(
        pl.kernel,
        mesh=mesh,
        out_type=jax.ShapeDtypeStruct((_B * EMBED_DIM,), jnp.float32),
        compiler_params=pltpu.CompilerParams(use_tc_tiling_on_sc=False),
        scratch_types=[
            pltpu.VMEM((_N_CHUNKS, _CHUNK), jnp.int32),
            pltpu.VMEM((_CHUNK, D_PAD), jnp.float32),
            pltpu.VMEM((_CHUNK, D_PAD), jnp.float32),
            pltpu.VMEM((_CHUNK_WORDS,), jnp.float32),
            pltpu.VMEM((_CHUNK_WORDS,), jnp.float32),
            pltpu.SemaphoreType.DMA,
            pltpu.SemaphoreType.DMA,
            pltpu.SemaphoreType.DMA,
            pltpu.SemaphoreType.DMA,
        ],
    )
    def k(idx_hbm, table_hbm, out_hbm, idx_v, buf0, buf1, cb0, cb1,
          sem0, sem1, osem0, osem1):
        wid = lax.axis_index("s") * 2 + lax.axis_index("c")
        obase = wid * _B_PER_W * EMBED_DIM

        pltpu.sync_copy(idx_hbm.at[wid], idx_v)

        bufs = (buf0, buf1)
        cbs = (cb0, cb1)
        sems = (sem0, sem1)
        osems = (osem0, osem1)

        pltpu.async_copy(table_hbm.at[idx_v.at[0]], buf0, sem0)

        def compact(src, dst):
            def row(r, carry):
                o = r * EMBED_DIM
                for kk in range(EMBED_DIM // _LANES):
                    v = src[r, pl.ds(kk * _LANES, _LANES)]
                    dst[pl.ds(o + kk * _LANES, _LANES)] = v
                v = src[r, pl.ds(EMBED_DIM - _LANES, _LANES)]
                dst[pl.ds(o + EMBED_DIM - _LANES, _LANES)] = v
                return carry

            lax.fori_loop(0, _CHUNK, row, 0, unroll=8)

        def outer(i, carry):
            for p in range(2):
                j = i * 2 + p
                cur, cur_sem = bufs[p], sems[p]
                cb, osem = cbs[p], osems[p]
                nxt, nxt_sem = bufs[1 - p], sems[1 - p]

                @pl.when(j + 1 < _N_CHUNKS)
                def _():
                    pltpu.async_copy(table_hbm.at[idx_v.at[j + 1]], nxt, nxt_sem)

                pltpu.make_async_copy(table_hbm.at[idx_v.at[j]], cur, cur_sem).wait()

                @pl.when(j >= 2)
                def _():
                    pltpu.make_async_copy(
                        cb, out_hbm.at[pl.ds(0, _CHUNK_WORDS)], osem
                    ).wait()

                compact(cur, cb)
                pltpu.async_copy(
                    cb,
                    out_hbm.at[pl.ds(obase + j * _CHUNK_WORDS, _CHUNK_WORDS)],
                    osem,
                )
            return carry

        lax.fori_loop(0, _N_CHUNKS // 2, outer, 0)

        pltpu.make_async_copy(cb0, out_hbm.at[pl.ds(0, _CHUNK_WORDS)], osem0).wait()
        pltpu.make_async_copy(cb1, out_hbm.at[pl.ds(0, _CHUNK_WORDS)], osem1).wait()

    return k


_sc_gather = _make_sc_gather()


def kernel(indices, table):
    idx = indices.reshape(_NW, _N_CHUNKS, _CHUNK).astype(jnp.int32)
    table_pad = jnp.pad(table, ((0, 0), (0, D_PAD - EMBED_DIM)))
    out = _sc_gather(idx, table_pad)
    return out.reshape(BATCH, SEQ_LEN, EMBED_DIM)
